# baseline (speedup 1.0000x reference)
import functools

import jax
import jax.numpy as jnp
from jax import lax
from jax.experimental import pallas as pl
from jax.experimental.pallas import tpu as pltpu
from jax.experimental.pallas import tpu_sc as plsc

NW = 32
CHUNK = 4
L = 16


def _sc_agg(table, nbr_flat, V, K, F2):
    rows = CHUNK * K
    nchunk = V // CHUNK
    base_cnt, extra = divmod(nchunk, NW)
    max_cnt = base_cnt + (1 if extra else 0)
    nj = F2 // L
    mesh = plsc.VectorSubcoreMesh(core_axis_name="c", subcore_axis_name="s")

    @functools.partial(
        pl.kernel,
        out_type=jax.ShapeDtypeStruct((V, F2), jnp.float32),
        mesh=mesh,
        scratch_types=[
            pltpu.VMEM((max_cnt * rows,), jnp.int32),
            pltpu.VMEM((rows, F2), jnp.float32),
            pltpu.VMEM((rows, F2), jnp.float32),
            pltpu.VMEM((CHUNK, F2), jnp.float32),
            pltpu.VMEM((CHUNK, F2), jnp.float32),
            pltpu.SemaphoreType.DMA,
            pltpu.SemaphoreType.DMA,
            pltpu.SemaphoreType.DMA,
            pltpu.SemaphoreType.DMA,
        ],
    )
    def agg(table_hbm, nbr_hbm, out_hbm, idx_v, rows0, rows1,
            out0, out1, gsem0, gsem1, osem0, osem1):
        wid = lax.axis_index("s") * 2 + lax.axis_index("c")
        cnt = base_cnt + jnp.where(wid < extra, 1, 0)
        first = wid * base_cnt + jnp.minimum(wid, extra)
        pltpu.sync_copy(nbr_hbm.at[pl.ds(first * rows, max_cnt * rows)],
                        idx_v)
        gbufs = ((rows0, gsem0), (rows1, gsem1))
        obufs = ((out0, osem0), (out1, osem1))

        def start(buf, t):
            rows_v, sem = buf
            pltpu.async_copy(
                table_hbm.at[idx_v.at[pl.ds(t * rows, rows)]], rows_v, sem)

        def finish(gbuf, obuf, t, drain_out):
            rows_v, sem = gbuf
            outrow_v, osem = obuf
            pltpu.make_async_copy(
                table_hbm.at[idx_v.at[pl.ds(t * rows, rows)]],
                rows_v, sem).wait()
            @pl.when(drain_out)
            def _():
                pltpu.make_async_copy(
                    outrow_v, out_hbm.at[pl.ds(0, CHUNK)], osem).wait()
            for n in range(CHUNK):
                base = n * K

                def quad(r):
                    return tuple(
                        (rows_v[r, pl.ds(j * L, L)]
                         + rows_v[r + 1, pl.ds(j * L, L)])
                        + (rows_v[r + 2, pl.ds(j * L, L)]
                           + rows_v[r + 3, pl.ds(j * L, L)])
                        for j in range(nj)
                    )

                def kbody(p, a):
                    q = quad(base + 4 * p)
                    return tuple(a[j] + q[j] for j in range(nj))

                accs = lax.fori_loop(1, K // 4, kbody, quad(base))
                scale = jnp.float32(1.0 / K)
                for j in range(nj):
                    outrow_v[n, pl.ds(j * L, L)] = accs[j] * scale
            pltpu.async_copy(
                outrow_v,
                out_hbm.at[pl.ds((first + t) * CHUNK, CHUNK)], osem)

        start(gbufs[0], 0)

        def pair_body(p, carry):
            t = p * 2
            start(gbufs[1], t + 1)
            finish(gbufs[0], obufs[0], t, p > 0)

            @pl.when(t + 2 < cnt)
            def _():
                start(gbufs[0], t + 2)

            finish(gbufs[1], obufs[1], t + 1, p > 0)
            return carry

        lax.fori_loop(0, cnt // 2, pair_body, 0)

        @pl.when(cnt % 2 == 1)
        def _():
            finish(gbufs[0], obufs[0], cnt - 1, cnt > 1)

        pltpu.make_async_copy(out0, out_hbm.at[pl.ds(0, CHUNK)],
                              osem0).wait()

        @pl.when(cnt >= 2)
        def _():
            pltpu.make_async_copy(out1, out_hbm.at[pl.ds(0, CHUNK)],
                                  osem1).wait()

    return agg(table, nbr_flat)


def _tc_transform(x, agg, Wx, Wn, bias, blk):
    B, V, F = x.shape

    def body(x_ref, a_ref, wx_ref, wn_ref, b_ref, o_ref):
        o = jnp.dot(x_ref[0], wx_ref[...], preferred_element_type=jnp.float32)
        o += jnp.dot(a_ref[...], wn_ref[...],
                     preferred_element_type=jnp.float32)
        o_ref[...] = (o + b_ref[...])[None]

    return pl.pallas_call(
        body,
        out_shape=jax.ShapeDtypeStruct((B, V, F), jnp.float32),
        grid=(B, V // blk),
        in_specs=[
            pl.BlockSpec((1, blk, F), lambda b, i: (b, i, 0)),
            pl.BlockSpec((blk, F), lambda b, i: (i, b)),
            pl.BlockSpec((F, F), lambda b, i: (0, 0)),
            pl.BlockSpec((F, F), lambda b, i: (0, 0)),
            pl.BlockSpec((1, F), lambda b, i: (0, 0)),
        ],
        out_specs=pl.BlockSpec((1, blk, F), lambda b, i: (b, i, 0)),
    )(x, agg, Wx, Wn, bias)


def kernel(x, neighbor, Wx, Wn, b):
    B, V, F = x.shape
    K = neighbor.shape[-1]
    table = jnp.transpose(x, (1, 0, 2)).reshape(V, B * F)
    table = jnp.concatenate([jnp.zeros((1, B * F), jnp.float32), table],
                            axis=0)
    nbr_flat = jnp.concatenate(
        [neighbor.reshape(-1),
         jnp.zeros((CHUNK * K,), jnp.int32)])
    agg = _sc_agg(table, nbr_flat, V, K, B * F)
    return _tc_transform(x, agg, Wx, Wn, b.reshape(1, F), 2000)


# speedup vs baseline: 1.4139x; 1.4139x over previous
import functools

import jax
import jax.numpy as jnp
from jax import lax
from jax.experimental import pallas as pl
from jax.experimental.pallas import tpu as pltpu
from jax.experimental.pallas import tpu_sc as plsc

NW = 32
CHUNK = 4
L = 16


def _sc_agg(table, nbr_flat, V, K, F2):
    rows = CHUNK * K
    nchunk = V // CHUNK
    base_cnt, extra = divmod(nchunk, NW)
    max_cnt = base_cnt + (1 if extra else 0)
    nj = F2 // L
    mesh = plsc.VectorSubcoreMesh(core_axis_name="c", subcore_axis_name="s")

    @functools.partial(
        pl.kernel,
        out_type=jax.ShapeDtypeStruct((V, F2), jnp.float32),
        mesh=mesh,
        scratch_types=[
            pltpu.VMEM((max_cnt * rows,), jnp.int32),
            pltpu.VMEM((rows, F2), jnp.float32),
            pltpu.VMEM((rows, F2), jnp.float32),
            pltpu.VMEM((CHUNK, F2), jnp.float32),
            pltpu.VMEM((CHUNK, F2), jnp.float32),
            pltpu.SemaphoreType.DMA,
            pltpu.SemaphoreType.DMA,
            pltpu.SemaphoreType.DMA,
            pltpu.SemaphoreType.DMA,
        ],
    )
    def agg(table_hbm, nbr_hbm, out_hbm, idx_v, rows0, rows1,
            out0, out1, gsem0, gsem1, osem0, osem1):
        wid = lax.axis_index("s") * 2 + lax.axis_index("c")
        cnt = base_cnt + jnp.where(wid < extra, 1, 0)
        first = wid * base_cnt + jnp.minimum(wid, extra)
        pltpu.sync_copy(nbr_hbm.at[pl.ds(first * rows, max_cnt * rows)],
                        idx_v)
        gbufs = ((rows0, gsem0), (rows1, gsem1))
        obufs = ((out0, osem0), (out1, osem1))

        half = rows // 2

        def start(buf, t):
            rows_v, sem = buf
            pltpu.async_copy(
                table_hbm.at[idx_v.at[pl.ds(t * rows, half)]],
                rows_v.at[pl.ds(0, half)], sem)
            pltpu.async_copy(
                table_hbm.at[idx_v.at[pl.ds(t * rows + half, half)]],
                rows_v.at[pl.ds(half, half)], sem)

        def finish(gbuf, obuf, t, drain_out):
            rows_v, sem = gbuf
            outrow_v, osem = obuf
            pltpu.make_async_copy(
                table_hbm.at[idx_v.at[pl.ds(t * rows, half)]],
                rows_v.at[pl.ds(0, half)], sem).wait()
            pltpu.make_async_copy(
                table_hbm.at[idx_v.at[pl.ds(t * rows + half, half)]],
                rows_v.at[pl.ds(half, half)], sem).wait()
            @pl.when(drain_out)
            def _():
                pltpu.make_async_copy(
                    outrow_v, out_hbm.at[pl.ds(0, CHUNK)], osem).wait()
            for n in range(CHUNK):
                base = n * K
                accs = tuple(
                    rows_v[base, pl.ds(j * L, L)]
                    + rows_v[base + 1, pl.ds(j * L, L)]
                    for j in range(nj)
                )

                def kbody(p, a):
                    k = base + 2 * p
                    return tuple(
                        a[j]
                        + rows_v[k, pl.ds(j * L, L)]
                        + rows_v[k + 1, pl.ds(j * L, L)]
                        for j in range(nj)
                    )

                accs = lax.fori_loop(1, K // 2, kbody, accs)
                scale = jnp.float32(1.0 / K)
                for j in range(nj):
                    outrow_v[n, pl.ds(j * L, L)] = accs[j] * scale
            pltpu.async_copy(
                outrow_v,
                out_hbm.at[pl.ds((first + t) * CHUNK, CHUNK)], osem)

        start(gbufs[0], 0)

        def pair_body(p, carry):
            t = p * 2
            start(gbufs[1], t + 1)
            finish(gbufs[0], obufs[0], t, p > 0)

            @pl.when(t + 2 < cnt)
            def _():
                start(gbufs[0], t + 2)

            finish(gbufs[1], obufs[1], t + 1, p > 0)
            return carry

        lax.fori_loop(0, cnt // 2, pair_body, 0)

        @pl.when(cnt % 2 == 1)
        def _():
            finish(gbufs[0], obufs[0], cnt - 1, cnt > 1)

        pltpu.make_async_copy(out0, out_hbm.at[pl.ds(0, CHUNK)],
                              osem0).wait()

        @pl.when(cnt >= 2)
        def _():
            pltpu.make_async_copy(out1, out_hbm.at[pl.ds(0, CHUNK)],
                                  osem1).wait()

    return agg(table, nbr_flat)


def _tc_transform(x, agg, Wx, Wn, bias, blk):
    B, V, F = x.shape

    def body(x_ref, a_ref, wx_ref, wn_ref, b_ref, o_ref):
        o = jnp.dot(x_ref[0], wx_ref[...], preferred_element_type=jnp.float32)
        o += jnp.dot(a_ref[...], wn_ref[...],
                     preferred_element_type=jnp.float32)
        o_ref[...] = (o + b_ref[...])[None]

    return pl.pallas_call(
        body,
        out_shape=jax.ShapeDtypeStruct((B, V, F), jnp.float32),
        grid=(B, V // blk),
        in_specs=[
            pl.BlockSpec((1, blk, F), lambda b, i: (b, i, 0)),
            pl.BlockSpec((blk, F), lambda b, i: (i, b)),
            pl.BlockSpec((F, F), lambda b, i: (0, 0)),
            pl.BlockSpec((F, F), lambda b, i: (0, 0)),
            pl.BlockSpec((1, F), lambda b, i: (0, 0)),
        ],
        out_specs=pl.BlockSpec((1, blk, F), lambda b, i: (b, i, 0)),
    )(x, agg, Wx, Wn, bias)


def kernel(x, neighbor, Wx, Wn, b):
    B, V, F = x.shape
    K = neighbor.shape[-1]
    table = jnp.transpose(x, (1, 0, 2)).reshape(V, B * F)
    table = jnp.concatenate([jnp.zeros((1, B * F), jnp.float32), table],
                            axis=0)
    nbr_flat = jnp.concatenate(
        [neighbor.reshape(-1),
         jnp.zeros((CHUNK * K,), jnp.int32)])
    agg = _sc_agg(table, nbr_flat, V, K, B * F)
    return _tc_transform(x, agg, Wx, Wn, b.reshape(1, F), 2000)


# speedup vs baseline: 1.4163x; 1.0017x over previous
import functools

import jax
import jax.numpy as jnp
from jax import lax
from jax.experimental import pallas as pl
from jax.experimental.pallas import tpu as pltpu
from jax.experimental.pallas import tpu_sc as plsc

NW = 32
CHUNK = 4
L = 16


def _sc_agg(table, nbr_flat, V, K, F2):
    rows = CHUNK * K
    nchunk = V // CHUNK
    base_cnt, extra = divmod(nchunk, NW)
    max_cnt = base_cnt + (1 if extra else 0)
    nj = F2 // L
    mesh = plsc.VectorSubcoreMesh(core_axis_name="c", subcore_axis_name="s")

    @functools.partial(
        pl.kernel,
        out_type=jax.ShapeDtypeStruct((V, F2), jnp.float32),
        mesh=mesh,
        scratch_types=[
            pltpu.VMEM((max_cnt * rows,), jnp.int32),
            pltpu.VMEM((rows, F2), jnp.float32),
            pltpu.VMEM((rows, F2), jnp.float32),
            pltpu.VMEM((CHUNK, F2), jnp.float32),
            pltpu.VMEM((CHUNK, F2), jnp.float32),
            pltpu.SemaphoreType.DMA,
            pltpu.SemaphoreType.DMA,
            pltpu.SemaphoreType.DMA,
            pltpu.SemaphoreType.DMA,
        ],
    )
    def agg(table_hbm, nbr_hbm, out_hbm, idx_v, rows0, rows1,
            out0, out1, gsem0, gsem1, osem0, osem1):
        wid = lax.axis_index("s") * 2 + lax.axis_index("c")
        cnt = base_cnt + jnp.where(wid < extra, 1, 0)
        first = wid * base_cnt + jnp.minimum(wid, extra)
        pltpu.sync_copy(nbr_hbm.at[pl.ds(first * rows, max_cnt * rows)],
                        idx_v)
        gbufs = ((rows0, gsem0), (rows1, gsem1))
        obufs = ((out0, osem0), (out1, osem1))

        def start(buf, t):
            rows_v, sem = buf
            pltpu.async_copy(
                table_hbm.at[idx_v.at[pl.ds(t * rows, rows)]], rows_v, sem)

        def finish(gbuf, obuf, t, drain_out):
            rows_v, sem = gbuf
            outrow_v, osem = obuf
            pltpu.make_async_copy(
                table_hbm.at[idx_v.at[pl.ds(t * rows, rows)]],
                rows_v, sem).wait()
            @pl.when(drain_out)
            def _():
                pltpu.make_async_copy(
                    outrow_v, out_hbm.at[pl.ds(0, CHUNK)], osem).wait()
            for n in range(CHUNK):
                base = n * K
                accs = tuple(
                    rows_v[base, pl.ds(j * L, L)]
                    + rows_v[base + 1, pl.ds(j * L, L)]
                    for j in range(nj)
                )

                def kbody(p, a):
                    k = base + 2 * p
                    return tuple(
                        a[j]
                        + rows_v[k, pl.ds(j * L, L)]
                        + rows_v[k + 1, pl.ds(j * L, L)]
                        for j in range(nj)
                    )

                accs = lax.fori_loop(1, K // 2, kbody, accs)
                scale = jnp.float32(1.0 / K)
                for j in range(nj):
                    outrow_v[n, pl.ds(j * L, L)] = accs[j] * scale
            pltpu.async_copy(
                outrow_v,
                out_hbm.at[pl.ds((first + t) * CHUNK, CHUNK)], osem)

        start(gbufs[0], 0)

        def pair_body(p, carry):
            t = p * 2
            start(gbufs[1], t + 1)
            finish(gbufs[0], obufs[0], t, p > 0)

            @pl.when(t + 2 < cnt)
            def _():
                start(gbufs[0], t + 2)

            finish(gbufs[1], obufs[1], t + 1, p > 0)
            return carry

        lax.fori_loop(0, cnt // 2, pair_body, 0)

        @pl.when(cnt % 2 == 1)
        def _():
            finish(gbufs[0], obufs[0], cnt - 1, cnt > 1)

        pltpu.make_async_copy(out0, out_hbm.at[pl.ds(0, CHUNK)],
                              osem0).wait()

        @pl.when(cnt >= 2)
        def _():
            pltpu.make_async_copy(out1, out_hbm.at[pl.ds(0, CHUNK)],
                                  osem1).wait()

    return agg(table, nbr_flat)


def _tc_transform(x, agg, Wx, Wn, bias, blk):
    B, V, F = x.shape

    def body(x_ref, a_ref, wx_ref, wn_ref, b_ref, o_ref):
        o = jnp.dot(x_ref[0], wx_ref[...], preferred_element_type=jnp.float32)
        o += jnp.dot(a_ref[...], wn_ref[...],
                     preferred_element_type=jnp.float32)
        o_ref[...] = (o + b_ref[...])[None]

    return pl.pallas_call(
        body,
        out_shape=jax.ShapeDtypeStruct((B, V, F), jnp.float32),
        grid=(B, V // blk),
        in_specs=[
            pl.BlockSpec((1, blk, F), lambda b, i: (b, i, 0)),
            pl.BlockSpec((blk, F), lambda b, i: (i, b)),
            pl.BlockSpec((F, F), lambda b, i: (0, 0)),
            pl.BlockSpec((F, F), lambda b, i: (0, 0)),
            pl.BlockSpec((1, F), lambda b, i: (0, 0)),
        ],
        out_specs=pl.BlockSpec((1, blk, F), lambda b, i: (b, i, 0)),
    )(x, agg, Wx, Wn, bias)


def kernel(x, neighbor, Wx, Wn, b):
    B, V, F = x.shape
    K = neighbor.shape[-1]
    table = jnp.transpose(x, (1, 0, 2)).reshape(V, B * F)
    table = jnp.concatenate([jnp.zeros((1, B * F), jnp.float32), table],
                            axis=0)
    nbr_flat = jnp.concatenate(
        [neighbor.reshape(-1),
         jnp.zeros((CHUNK * K,), jnp.int32)])
    agg = _sc_agg(table, nbr_flat, V, K, B * F)
    return _tc_transform(x, agg, Wx, Wn, b.reshape(1, F), 2000)
